# baseline (speedup 1.0000x reference)
import functools

import jax
import jax.numpy as jnp
from jax import lax
from jax.experimental import pallas as pl
from jax.experimental.pallas import tpu as pltpu
from jax.experimental.pallas import tpu_sc as plsc

_V = 21
_D = 1024
_B = 4 * 8192
_NW = 32
_BPW = _B // _NW
_LAG = 32

_mesh = plsc.VectorSubcoreMesh(core_axis_name="c", subcore_axis_name="s")


@functools.partial(
    pl.kernel,
    mesh=_mesh,
    out_type=jax.ShapeDtypeStruct((_B, _D), jnp.float32),
    scratch_types=[
        pltpu.VMEM((_BPW,), jnp.int32),
        pltpu.VMEM((_V * _D,), jnp.float32),
        pltpu.VMEM_SHARED((_V * _D,), jnp.float32),
        pltpu.SemaphoreType.DMA,
        pltpu.SemaphoreType.DMA,
    ],
)
def _emb_lookup(idx_hbm, table_hbm, out_hbm, idx_v, table_v, table_s, sem,
                sem_idx):
    sid = lax.axis_index("s")
    wid = sid * 2 + lax.axis_index("c")
    base = wid * _BPW
    idx_cp = pltpu.make_async_copy(idx_hbm.at[wid], idx_v, sem_idx)
    idx_cp.start()

    @pl.when(sid == 0)
    def _stage_table():
        pltpu.sync_copy(table_hbm, table_s)

    plsc.subcore_barrier()
    pltpu.sync_copy(table_s, table_v)
    idx_cp.wait()

    def row_copy(d, i):
        return pltpu.make_async_copy(
            table_v.at[pl.ds(d * _D, _D)], out_hbm.at[base + i], sem)

    def drain_one():
        row_copy(0, 0).wait()

    def body(g, _):
        dvec = idx_v[pl.ds(g * _LAG, _LAG)]
        for r in range(_LAG):
            row_copy(dvec[r], g * _LAG + r).start()

        @pl.when(g > 0)
        def _():
            for _r in range(_LAG):
                drain_one()

        return ()

    lax.fori_loop(0, _BPW // _LAG, body, (), unroll=False)

    def tail(i, _):
        drain_one()
        return ()

    lax.fori_loop(0, _LAG, tail, (), unroll=False)


def kernel(structural_position_ids, weight):
    ids = structural_position_ids.reshape(_NW, _BPW).astype(jnp.int32)
    out = _emb_lookup(ids, weight.reshape(-1))
    return out.reshape(structural_position_ids.shape + (_D,))


# speedup vs baseline: 1.0023x; 1.0023x over previous
import functools

import jax
import jax.numpy as jnp
from jax import lax
from jax.experimental import pallas as pl
from jax.experimental.pallas import tpu as pltpu
from jax.experimental.pallas import tpu_sc as plsc

_V = 21
_D = 1024
_B = 4 * 8192
_NW = 32
_BPW = _B // _NW
_LAG = 16

_mesh = plsc.VectorSubcoreMesh(core_axis_name="c", subcore_axis_name="s")


@functools.partial(
    pl.kernel,
    mesh=_mesh,
    out_type=jax.ShapeDtypeStruct((_B, _D), jnp.float32),
    scratch_types=[
        pltpu.VMEM((_BPW,), jnp.int32),
        pltpu.VMEM((_V * _D,), jnp.float32),
        pltpu.VMEM_SHARED((_V * _D,), jnp.float32),
        pltpu.SemaphoreType.DMA,
        pltpu.SemaphoreType.DMA,
    ],
)
def _emb_lookup(idx_hbm, table_hbm, out_hbm, idx_v, table_v, table_s, sem,
                sem_idx):
    sid = lax.axis_index("s")
    wid = sid * 2 + lax.axis_index("c")
    base = wid * _BPW
    idx_cp = pltpu.make_async_copy(idx_hbm.at[wid], idx_v, sem_idx)
    idx_cp.start()

    @pl.when(sid == 0)
    def _stage_table():
        pltpu.sync_copy(table_hbm, table_s)

    plsc.subcore_barrier()
    pltpu.sync_copy(table_s, table_v)
    idx_cp.wait()

    def row_copy(d, i):
        return pltpu.make_async_copy(
            table_v.at[pl.ds(d * _D, _D)], out_hbm.at[base + i], sem)

    def drain_one():
        row_copy(0, 0).wait()

    def body(g, _):
        dvec = idx_v[pl.ds(g * _LAG, _LAG)]
        for r in range(_LAG):
            row_copy(dvec[r], g * _LAG + r).start()

        @pl.when(g > 0)
        def _():
            for _r in range(_LAG):
                drain_one()

        return ()

    lax.fori_loop(0, _BPW // _LAG, body, (), unroll=False)

    def tail(i, _):
        drain_one()
        return ()

    lax.fori_loop(0, _LAG, tail, (), unroll=False)


def kernel(structural_position_ids, weight):
    ids = structural_position_ids.reshape(_NW, _BPW).astype(jnp.int32)
    out = _emb_lookup(ids, weight.reshape(-1))
    return out.reshape(structural_position_ids.shape + (_D,))
